# baseline (speedup 1.0000x reference)
import functools

import jax
import jax.numpy as jnp
from jax import lax
from jax.experimental import pallas as pl
from jax.experimental.pallas import tpu as pltpu
from jax.experimental.pallas import tpu_sc as plsc

_T, _C, _D, _H, _W = 8, 32, 64, 64, 64
_V = _D * _H * _W
_N = 262144

_NC = 2
_NS = 16
_NW = _NC * _NS
_L = 16

_CH = 256
_NPW = _N // _NW
_NCHUNK = _NPW // _CH
_GROWS = 128
_G = 8 * _CH // _GROWS

_CORNER_OFFS = (0, 1, _W, _W + 1, _H * _W, _H * _W + 1, _H * _W + _W, _H * _W + _W + 1)


def _rne_bf16_bits(v):
    b = lax.bitcast_convert_type(v, jnp.int32)
    return b + 0x7FFF + lax.bitwise_and(lax.shift_right_logical(b, 16), 1)


def _blend_body(t_ref, fa_ref, fb_ref, w_ref, x_ref, out_ref, xp_ref):
    fa = fa_ref[0, :, 0].reshape(_C, _H * _W)
    fb = fb_ref[0, :, 0].reshape(_C, _H * _W)
    blended = fa * w_ref[0] + fb * w_ref[1]
    tr = jnp.transpose(blended)
    rl = _rne_bf16_bits(tr[:, :_C // 2])
    rh = _rne_bf16_bits(tr[:, _C // 2:])
    out_ref[...] = jnp.bitwise_or(
        lax.shift_right_logical(rl, 16),
        jnp.bitwise_and(rh, jnp.int32(-65536)))
    xp_ref[...] = jnp.transpose(x_ref[...])


def _blend(features, tvec, w, x):
    xblk = _N // _D
    return pl.pallas_call(
        _blend_body,
        grid_spec=pltpu.PrefetchScalarGridSpec(
            num_scalar_prefetch=1,
            grid=(_D,),
            in_specs=[
                pl.BlockSpec((1, _C, 1, _H, _W), lambda j, t: (t[0], 0, j, 0, 0)),
                pl.BlockSpec((1, _C, 1, _H, _W), lambda j, t: (t[1], 0, j, 0, 0)),
                pl.BlockSpec(memory_space=pltpu.SMEM),
                pl.BlockSpec((xblk, 3), lambda j, t: (j, 0)),
            ],
            out_specs=[
                pl.BlockSpec((_H * _W, _C // 2), lambda j, t: (j, 0)),
                pl.BlockSpec((3, xblk), lambda j, t: (0, j)),
            ],
        ),
        out_shape=[jax.ShapeDtypeStruct((_V, _C // 2), jnp.int32),
                   jax.ShapeDtypeStruct((3, _N), jnp.float32)],
    )(tvec, features, features, w, x)


@functools.partial(
    pl.kernel,
    mesh=plsc.VectorSubcoreMesh(core_axis_name="c", subcore_axis_name="s"),
    out_type=jax.ShapeDtypeStruct((_N // 4, 4 * _C), jnp.float32),
    compiler_params=pltpu.CompilerParams(use_tc_tiling_on_sc=False,
                                         needs_layout_passes=False),
    scratch_types=[
        pltpu.VMEM((3, _CH), jnp.float32),
        pltpu.VMEM((3 * _CH + _L,), jnp.float32),
        pltpu.VMEM((_G, _GROWS), jnp.int32),
        pltpu.VMEM((8 * _CH, _C // 2), jnp.int32),
        pltpu.VMEM((_CH // 4, 4 * _C), jnp.float32),
        pltpu.SemaphoreType.DMA,
    ],
)
def _sc_sample(xt_hbm, table_hbm, out_hbm,
               xt_v, t_v, idx_v, rows_v, out_v, sem):
    cid = lax.axis_index("c")
    sid = lax.axis_index("s")
    wid = sid * _NC + cid

    def chunk(ci, carry):
        base = pl.multiple_of(wid * _NPW + ci * _CH, _CH)
        pltpu.sync_copy(xt_hbm.at[:, pl.ds(base, _CH)], xt_v)

        for g in range(_CH // _L):
            vr = []
            for r in range(3):
                u = jnp.minimum(
                    jnp.maximum(xt_v[r, pl.ds(g * _L, _L)] * 64.0 - 0.5,
                                0.0), 63.0)
                i0 = jnp.minimum(u.astype(jnp.int32), 62)
                t_v[pl.ds(r * _CH + g * _L, _L)] = u - i0.astype(jnp.float32)
                vr.append(i0)

            b = vr[0] + vr[1] * _W + vr[2] * (_H * _W)
            for k, off in enumerate(_CORNER_OFFS):
                pos = k * _CH + g * _L
                idx_v[pos // _GROWS, pl.ds(pos % _GROWS, _L)] = b + off

        copies = [
            pltpu.async_copy(
                table_hbm.at[idx_v.at[j]],
                rows_v.at[pl.ds(j * _GROWS, _GROWS)],
                sem,
            )
            for j in range(_G)
        ]
        for cp in copies:
            cp.wait()

        def pt(p, acc):
            def wvec(off):
                tf = jnp.full((_L,), t_v[pl.ds(off * _CH + p, _L)][0],
                              jnp.float32)
                return plsc.pack(tf, tf, format=plsc.PackFormat.INTERLEAVED)

            txv, tyv, tzv = wvec(0), wvec(1), wvec(2)
            v = [plsc.bitcast(rows_v[k * _CH + p, :], jnp.bfloat16)
                 for k in range(8)]
            cx00 = v[0] + txv * (v[1] - v[0])
            cx01 = v[2] + txv * (v[3] - v[2])
            cx10 = v[4] + txv * (v[5] - v[4])
            cx11 = v[6] + txv * (v[7] - v[6])
            c0 = cx00 + tyv * (cx01 - cx00)
            c1 = cx10 + tyv * (cx11 - cx10)
            res = c0 + tzv * (c1 - c0)
            lo, hi = plsc.unpack(res, format=plsc.PackFormat.INTERLEAVED)
            prow = lax.shift_right_logical(p, 2)
            pcol = lax.bitwise_and(p, 3) * _C
            out_v[prow, pl.ds(pcol, _L)] = lo
            out_v[prow, pl.ds(pcol + _L, _L)] = hi
            return acc

        lax.fori_loop(0, _CH, pt, 0)
        pltpu.sync_copy(out_v, out_hbm.at[pl.ds(base // 4, _CH // 4)])
        return carry

    lax.fori_loop(0, _NCHUNK, chunk, 0)


def _finalize_body(in_ref, out_ref):
    xb = in_ref[...]
    row = lax.broadcasted_iota(jnp.int32, (4 * _C, _C), 0)
    col = lax.broadcasted_iota(jnp.int32, (4 * _C, _C), 1)
    parts = []
    for j in range(4):
        sel = (row == col + j * _C).astype(jnp.float32)
        parts.append(jax.lax.dot(xb, sel,
                                 preferred_element_type=jnp.float32))
    st = jnp.stack(parts, axis=1)
    out_ref[...] = st.reshape(-1, _C)


def _finalize(out4):
    blk4 = 2048
    return pl.pallas_call(
        _finalize_body,
        grid=(_N // 4 // blk4,),
        in_specs=[pl.BlockSpec((blk4, 4 * _C), lambda j: (j, 0))],
        out_specs=pl.BlockSpec((4 * blk4, _C), lambda j: (j, 0)),
        out_shape=jax.ShapeDtypeStruct((_N, _C), jnp.float32),
    )(out4)


def kernel(x, idx, features):
    num_t = features.shape[0]
    idx_val = idx.reshape(())
    t0 = jnp.clip(jnp.floor(idx_val).astype(jnp.int32), 0, num_t - 1)
    t1 = jnp.minimum(t0 + 1, num_t - 1)
    f = idx_val - t0.astype(jnp.float32)

    w = jnp.stack([1.0 - f, f])
    tvec = jnp.stack([t0, t1])

    table, xt = _blend(features, tvec, w, x)
    return _finalize(_sc_sample(xt, table))


# speedup vs baseline: 1.1046x; 1.1046x over previous
import functools

import jax
import jax.numpy as jnp
from jax import lax
from jax.experimental import pallas as pl
from jax.experimental.pallas import tpu as pltpu
from jax.experimental.pallas import tpu_sc as plsc

_T, _C, _D, _H, _W = 8, 32, 64, 64, 64
_V = _D * _H * _W
_N = 262144

_NC = 2
_NS = 16
_NW = _NC * _NS
_L = 16

_CH = 256
_NPW = _N // _NW
_NCHUNK = _NPW // _CH
_GROWS = 128
_G = 8 * _CH // _GROWS

_CORNER_OFFS = (0, 1, _W, _W + 1, _H * _W, _H * _W + 1, _H * _W + _W, _H * _W + _W + 1)


def _blend_body(t_ref, fa_ref, fb_ref, w_ref, x_ref, out_ref, xp_ref):
    fa = fa_ref[0, :, 0].reshape(_C, _H * _W)
    fb = fb_ref[0, :, 0].reshape(_C, _H * _W)
    blended = fa * w_ref[0] + fb * w_ref[1]
    out_ref[...] = jnp.transpose(blended).astype(jnp.bfloat16)
    xp_ref[...] = jnp.transpose(x_ref[...])


def _blend(features, tvec, w, x):
    xblk = _N // _D
    return pl.pallas_call(
        _blend_body,
        grid_spec=pltpu.PrefetchScalarGridSpec(
            num_scalar_prefetch=1,
            grid=(_D,),
            in_specs=[
                pl.BlockSpec((1, _C, 1, _H, _W), lambda j, t: (t[0], 0, j, 0, 0)),
                pl.BlockSpec((1, _C, 1, _H, _W), lambda j, t: (t[1], 0, j, 0, 0)),
                pl.BlockSpec(memory_space=pltpu.SMEM),
                pl.BlockSpec((xblk, 3), lambda j, t: (j, 0)),
            ],
            out_specs=[
                pl.BlockSpec((_H * _W, _C), lambda j, t: (j, 0)),
                pl.BlockSpec((3, xblk), lambda j, t: (0, j)),
            ],
        ),
        out_shape=[jax.ShapeDtypeStruct((_V, _C), jnp.bfloat16),
                   jax.ShapeDtypeStruct((3, _N), jnp.float32)],
    )(tvec, features, features, w, x)


@functools.partial(
    pl.kernel,
    mesh=plsc.VectorSubcoreMesh(core_axis_name="c", subcore_axis_name="s"),
    out_type=jax.ShapeDtypeStruct((_N, _C), jnp.bfloat16),
    compiler_params=pltpu.CompilerParams(use_tc_tiling_on_sc=False,
                                         needs_layout_passes=False),
    scratch_types=[
        pltpu.VMEM((3, _CH), jnp.float32),
        pltpu.VMEM((3 * _CH + _L,), jnp.float32),
        pltpu.VMEM((_G, _GROWS), jnp.int32),
        pltpu.VMEM((8 * _CH, _C), jnp.bfloat16),
        pltpu.VMEM((_CH, _C), jnp.bfloat16),
        pltpu.SemaphoreType.DMA,
    ],
)
def _sc_sample(xt_hbm, table_hbm, out_hbm,
               xt_v, t_v, idx_v, rows_v, out_v, sem):
    cid = lax.axis_index("c")
    sid = lax.axis_index("s")
    wid = sid * _NC + cid

    def chunk(ci, carry):
        base = pl.multiple_of(wid * _NPW + ci * _CH, _CH)
        pltpu.sync_copy(xt_hbm.at[:, pl.ds(base, _CH)], xt_v)

        for g in range(_CH // _L):
            vr = []
            for r in range(3):
                u = jnp.minimum(
                    jnp.maximum(xt_v[r, pl.ds(g * _L, _L)] * 64.0 - 0.5,
                                0.0), 63.0)
                i0 = jnp.minimum(u.astype(jnp.int32), 62)
                t_v[pl.ds(r * _CH + g * _L, _L)] = u - i0.astype(jnp.float32)
                vr.append(i0)

            b = vr[0] + vr[1] * _W + vr[2] * (_H * _W)
            for k, off in enumerate(_CORNER_OFFS):
                pos = k * _CH + g * _L
                idx_v[pos // _GROWS, pl.ds(pos % _GROWS, _L)] = b + off

        copies = [
            pltpu.async_copy(
                table_hbm.at[idx_v.at[j]],
                rows_v.at[pl.ds(j * _GROWS, _GROWS)],
                sem,
            )
            for j in range(_G)
        ]
        for cp in copies:
            cp.wait()

        def pt(p, acc):
            def wvec(off):
                tf = jnp.full((_L,), t_v[pl.ds(off * _CH + p, _L)][0],
                              jnp.float32)
                return plsc.pack(tf, tf, format=plsc.PackFormat.INTERLEAVED)

            txv, tyv, tzv = wvec(0), wvec(1), wvec(2)
            v = [rows_v[k * _CH + p, :] for k in range(8)]
            cx00 = v[0] + txv * (v[1] - v[0])
            cx01 = v[2] + txv * (v[3] - v[2])
            cx10 = v[4] + txv * (v[5] - v[4])
            cx11 = v[6] + txv * (v[7] - v[6])
            c0 = cx00 + tyv * (cx01 - cx00)
            c1 = cx10 + tyv * (cx11 - cx10)
            out_v[p, :] = c0 + tzv * (c1 - c0)
            return acc

        lax.fori_loop(0, _CH, pt, 0)
        pltpu.sync_copy(out_v, out_hbm.at[pl.ds(base, _CH)])
        return carry

    lax.fori_loop(0, _NCHUNK, chunk, 0)


def kernel(x, idx, features):
    num_t = features.shape[0]
    idx_val = idx.reshape(())
    t0 = jnp.clip(jnp.floor(idx_val).astype(jnp.int32), 0, num_t - 1)
    t1 = jnp.minimum(t0 + 1, num_t - 1)
    f = idx_val - t0.astype(jnp.float32)

    w = jnp.stack([1.0 - f, f])
    tvec = jnp.stack([t0, t1])

    table, xt = _blend(features, tvec, w, x)
    return _sc_sample(xt, table).astype(jnp.float32)


# speedup vs baseline: 1.2541x; 1.1354x over previous
import functools

import jax
import jax.numpy as jnp
from jax import lax
from jax.experimental import pallas as pl
from jax.experimental.pallas import tpu as pltpu
from jax.experimental.pallas import tpu_sc as plsc

_T, _C, _D, _H, _W = 8, 32, 64, 64, 64
_V = _D * _H * _W
_N = 262144

_NC = 2
_NS = 16
_NW = _NC * _NS
_L = 16

_CH = 256
_NPW = _N // _NW
_NCHUNK = _NPW // _CH
_GROWS = 128
_G = 8 * _CH // _GROWS

_CORNER_OFFS = (0, 1, _W, _W + 1, _H * _W, _H * _W + 1, _H * _W + _W, _H * _W + _W + 1)


def _blend_body(t_ref, fa_ref, fb_ref, w_ref, x_ref, out_ref, xp_ref):
    fa = fa_ref[0, :, 0].reshape(_C, _H * _W)
    fb = fb_ref[0, :, 0].reshape(_C, _H * _W)
    blended = fa * w_ref[0] + fb * w_ref[1]
    out_ref[...] = jnp.transpose(blended).astype(jnp.bfloat16)
    xp_ref[...] = jnp.transpose(x_ref[...])


def _blend(features, tvec, w, x):
    xblk = _N // _D
    return pl.pallas_call(
        _blend_body,
        grid_spec=pltpu.PrefetchScalarGridSpec(
            num_scalar_prefetch=1,
            grid=(_D,),
            in_specs=[
                pl.BlockSpec((1, _C, 1, _H, _W), lambda j, t: (t[0], 0, j, 0, 0)),
                pl.BlockSpec((1, _C, 1, _H, _W), lambda j, t: (t[1], 0, j, 0, 0)),
                pl.BlockSpec(memory_space=pltpu.SMEM),
                pl.BlockSpec((xblk, 3), lambda j, t: (j, 0)),
            ],
            out_specs=[
                pl.BlockSpec((_H * _W, _C), lambda j, t: (j, 0)),
                pl.BlockSpec((3, xblk), lambda j, t: (0, j)),
            ],
        ),
        out_shape=[jax.ShapeDtypeStruct((_V, _C), jnp.bfloat16),
                   jax.ShapeDtypeStruct((3, _N), jnp.float32)],
    )(tvec, features, features, w, x)


@functools.partial(
    pl.kernel,
    mesh=plsc.VectorSubcoreMesh(core_axis_name="c", subcore_axis_name="s"),
    out_type=jax.ShapeDtypeStruct((_N, _C), jnp.bfloat16),
    compiler_params=pltpu.CompilerParams(use_tc_tiling_on_sc=False,
                                         needs_layout_passes=False),
    scratch_types=[
        pltpu.VMEM((2, 3, _CH), jnp.float32),
        pltpu.VMEM((2, 3 * _CH + _L), jnp.float32),
        pltpu.VMEM((2, _G, _GROWS), jnp.int32),
        pltpu.VMEM((2, 8 * _CH, _C), jnp.bfloat16),
        pltpu.VMEM((_CH, _C), jnp.bfloat16),
        pltpu.SemaphoreType.DMA,
        pltpu.SemaphoreType.DMA,
    ],
)
def _sc_sample(xt_hbm, table_hbm, out_hbm,
               xt_v, t_v, idx_v, rows_v, out_v, sem_a, sem_b):
    cid = lax.axis_index("c")
    sid = lax.axis_index("s")
    wid = sid * _NC + cid
    sems = (sem_a, sem_b)

    def phase1_fire(ci, bi):
        base = pl.multiple_of(wid * _NPW + ci * _CH, _CH)
        pltpu.sync_copy(xt_hbm.at[:, pl.ds(base, _CH)], xt_v.at[bi])
        for g in range(_CH // _L):
            vr = []
            for r in range(3):
                u = jnp.minimum(
                    jnp.maximum(xt_v[bi, r, pl.ds(g * _L, _L)] * 64.0 - 0.5,
                                0.0), 63.0)
                i0 = jnp.minimum(u.astype(jnp.int32), 62)
                t_v[bi, pl.ds(r * _CH + g * _L, _L)] = u - i0.astype(jnp.float32)
                vr.append(i0)

            b = vr[0] + vr[1] * _W + vr[2] * (_H * _W)
            for k, off in enumerate(_CORNER_OFFS):
                pos = k * _CH + g * _L
                idx_v[bi, pos // _GROWS, pl.ds(pos % _GROWS, _L)] = b + off

        for j in range(_G):
            pltpu.async_copy(
                table_hbm.at[idx_v.at[bi, j]],
                rows_v.at[bi, pl.ds(j * _GROWS, _GROWS)],
                sems[bi],
            )

    def combine_store(ci, bi):
        base = pl.multiple_of(wid * _NPW + ci * _CH, _CH)
        pltpu.make_async_copy(
            table_hbm.at[pl.ds(0, 8 * _CH)], rows_v.at[bi], sems[bi],
        ).wait()

        def pt(p, acc):
            def wvec(off):
                tf = jnp.full((_L,), t_v[bi, pl.ds(off * _CH + p, _L)][0],
                              jnp.float32)
                return plsc.pack(tf, tf, format=plsc.PackFormat.INTERLEAVED)

            txv, tyv, tzv = wvec(0), wvec(1), wvec(2)
            v = [rows_v[bi, k * _CH + p, :] for k in range(8)]
            cx00 = v[0] + txv * (v[1] - v[0])
            cx01 = v[2] + txv * (v[3] - v[2])
            cx10 = v[4] + txv * (v[5] - v[4])
            cx11 = v[6] + txv * (v[7] - v[6])
            c0 = cx00 + tyv * (cx01 - cx00)
            c1 = cx10 + tyv * (cx11 - cx10)
            out_v[p, :] = c0 + tzv * (c1 - c0)
            return acc

        lax.fori_loop(0, _CH, pt, 0)
        pltpu.sync_copy(out_v, out_hbm.at[pl.ds(base, _CH)])

    phase1_fire(0, 0)

    def pair(i, carry):
        ci0 = 2 * i
        for b in range(2):
            ci = ci0 + b
            nxt = ci + 1

            @pl.when(nxt < _NCHUNK)
            def _():
                phase1_fire(nxt, (b + 1) % 2)

            combine_store(ci, b)
        return carry

    lax.fori_loop(0, _NCHUNK // 2, pair, 0)


def kernel(x, idx, features):
    num_t = features.shape[0]
    idx_val = idx.reshape(())
    t0 = jnp.clip(jnp.floor(idx_val).astype(jnp.int32), 0, num_t - 1)
    t1 = jnp.minimum(t0 + 1, num_t - 1)
    f = idx_val - t0.astype(jnp.float32)

    w = jnp.stack([1.0 - f, f])
    tvec = jnp.stack([t0, t1])

    table, xt = _blend(features, tvec, w, x)
    return _sc_sample(xt, table).astype(jnp.float32)
